# baseline (speedup 1.0000x reference)
import functools

import jax
import jax.numpy as jnp
from jax import lax
from jax.experimental import pallas as pl
from jax.experimental.pallas import tpu as pltpu
from jax.experimental.pallas import tpu_sc as plsc

_B = 32
_N = 192 * 56 * 56
_K = max(1, int(_N * 0.2))
_R = _N // 128
_HW = 56 * 56
_C = 192
_CPW = _C // 16
_CHR = 32
_NPAIR = _HW // (2 * _CHR)
_SURV = 24576
_SIGN = 0x7FFFFFFF
_NC = 2
_NS = 16


def _sc_thresholds_kernel(
    x_hbm, thr_hbm, buf0, buf1, hist, surv, thrv, gs, sem0, sem1
):
    wid = lax.axis_index("s") * _NC + lax.axis_index("c")
    lane = lax.iota(jnp.int32, 16)
    ones = jnp.ones((16,), jnp.int32)

    def zero_hist(nwords):
        @plsc.parallel_loop(0, nwords, step=16, unroll=8)
        def _(j):
            hist[pl.ds(j, 16)] = jnp.zeros((16,), jnp.int32)

    def stream_pass(process, carry0):
        pltpu.async_copy(x_hbm.at[wid, pl.ds(0, _CHR)], buf0, sem0)
        pltpu.async_copy(x_hbm.at[wid, pl.ds(_CHR, _CHR)], buf1, sem1)

        def pbody(i, carry):
            nxt = (2 * i + 2) * _CHR
            pltpu.make_async_copy(x_hbm.at[wid, pl.ds(0, _CHR)], buf0, sem0).wait()
            carry = process(buf0, carry)

            @pl.when(i < _NPAIR - 1)
            def _():
                pltpu.async_copy(x_hbm.at[wid, pl.ds(nxt, _CHR)], buf0, sem0)

            pltpu.make_async_copy(x_hbm.at[wid, pl.ds(0, _CHR)], buf1, sem1).wait()
            carry = process(buf1, carry)

            @pl.when(i < _NPAIR - 1)
            def _():
                pltpu.async_copy(x_hbm.at[wid, pl.ds(nxt + _CHR, _CHR)], buf1, sem1)

            return carry

        return lax.fori_loop(0, _NPAIR, pbody, carry0)

    def hist_pass(make_idx_mask):
        def process(buf, carry):
            @plsc.parallel_loop(0, _CHR, step=1, unroll=2)
            def _(r):
                for c in range(_CPW):
                    v = buf[r, pl.ds(c * 16, 16)]
                    t = plsc.bitcast(v, jnp.int32) & _SIGN
                    idx, m = make_idx_mask(t)
                    plsc.addupdate_scatter(hist, [idx], ones, mask=m)

            return carry

        stream_pass(process, 0)

    def select(ngroups, rank):
        def gbody(g, c):
            acc = hist[pl.ds(g * 256, 16)]
            for l in range(1, 16):
                acc = acc + hist[pl.ds(g * 256 + l * 16, 16)]
            gs[g] = jnp.sum(acc)
            return c

        lax.fori_loop(0, ngroups, gbody, 0)

        def sbody(i, carry):
            acc, grp, s_after = carry
            g = ngroups - 1 - i
            nacc = acc + gs[g]
            hit = jnp.logical_and(nacc >= rank, acc < rank)
            return (
                nacc,
                jnp.where(hit, g, grp),
                jnp.where(hit, acc, s_after),
            )

        _, grp, s_after = lax.fori_loop(
            0, ngroups, sbody, (jnp.int32(0), jnp.int32(0), jnp.int32(0))
        )

        tv = jnp.zeros((16,), jnp.int32)
        gbase = grp * 256
        for l in range(16):
            tv = tv + plsc.load_gather(hist, [gbase + lane * 16 + l])

        rr = rank - s_after
        rv = lax.rev(tv, (0,))
        cum = plsc.cumsum(rv)
        i0 = plsc.all_reduce_ffs(cum >= rr)
        sel = lane == i0
        rv_i = jnp.sum(jnp.where(sel, rv, 0))
        c_i = jnp.sum(jnp.where(sel, cum, 0))
        binlow = 15 - jnp.sum(jnp.where(sel, lane, 0))
        b = grp * 16 + binlow
        return b, rr - (c_i - rv_i), rv_i

    zero_hist(65536)

    def l1(t):
        return ((t >> 15) & 0x1FFF0) | lane, None

    hist_pass(l1)
    b1, r2, c1 = select(256, jnp.int32(_K))

    zero_hist(65536)

    def process_l2(buf, off):
        @plsc.parallel_loop(0, _CHR, step=1, unroll=2, carry=off)
        def body(r, off):
            for c in range(_CPW):
                v = buf[r, pl.ds(c * 16, 16)]
                t = plsc.bitcast(v, jnp.int32) & _SIGN
                m = (t >> 19) == b1
                idx = ((t >> 3) & 0xFFF0) | lane
                plsc.addupdate_scatter(hist, [idx], ones, mask=m)
                pos = plsc.cumsum(m.astype(jnp.int32))
                sidx = jnp.minimum(off + pos - 1, _SURV - 1)
                plsc.store_scatter(surv, [sidx], t, mask=m)
                off = off + plsc.all_reduce_population_count(m)
            return off

        return body

    stream_pass(process_l2, jnp.zeros((16,), jnp.int32))
    b2, r3, _ = select(256, r2)

    p25 = (b1 << 12) | b2
    zero_hist(2048)

    def l3(t):
        return ((t & 0x7F) << 4) | lane, (t >> 7) == p25

    @pl.when(c1 <= _SURV)
    def _():
        nv = (c1 + 15) // 16

        def vb(j, carry):
            t = surv[pl.ds(j * 16, 16)]
            m = jnp.logical_and((j * 16 + lane) < c1, (t >> 7) == p25)
            plsc.addupdate_scatter(hist, [((t & 0x7F) << 4) | lane], ones, mask=m)
            return carry

        lax.fori_loop(0, nv, vb, 0)

    @pl.when(c1 > _SURV)
    def _():
        hist_pass(l3)

    b3, _, _ = select(8, r3)

    bits = (b1 << 19) | (b2 << 7) | b3
    thrv[...] = jnp.full((16,), 1, jnp.int32) * bits
    pltpu.sync_copy(thrv, thr_hbm.at[wid])


def _mask_kernel(thr_ref, x_ref, o_ref):
    b = thr_ref[pl.program_id(0)]
    xb = x_ref[0]
    bits = lax.bitcast_convert_type(xb, jnp.int32) & _SIGN
    o_ref[0] = jnp.where(bits >= b, xb, 0.0)


def kernel(x):
    hw = x.shape[2] * x.shape[3]
    ch = x.shape[1]
    xt3 = jnp.transpose(x, (0, 2, 3, 1)).reshape(_B, hw, ch)

    mesh = plsc.VectorSubcoreMesh(core_axis_name="c", subcore_axis_name="s")
    thr2d = pl.kernel(
        _sc_thresholds_kernel,
        out_type=jax.ShapeDtypeStruct((_B, 16), jnp.int32),
        mesh=mesh,
        compiler_params=pltpu.CompilerParams(needs_layout_passes=False),
        scratch_types=[
            pltpu.VMEM((_CHR, _C), jnp.float32),
            pltpu.VMEM((_CHR, _C), jnp.float32),
            pltpu.VMEM((65536,), jnp.int32),
            pltpu.VMEM((_SURV,), jnp.int32),
            pltpu.VMEM((16,), jnp.int32),
            pltpu.SMEM((256,), jnp.int32),
            pltpu.SemaphoreType.DMA,
            pltpu.SemaphoreType.DMA,
        ],
    )(xt3)
    out3 = pl.pallas_call(
        _mask_kernel,
        grid=(_B,),
        in_specs=[
            pl.BlockSpec(memory_space=pltpu.SMEM),
            pl.BlockSpec((1, hw, ch), lambda i: (i, 0, 0)),
        ],
        out_specs=pl.BlockSpec((1, hw, ch), lambda i: (i, 0, 0)),
        out_shape=jax.ShapeDtypeStruct((_B, hw, ch), jnp.float32),
    )(thr2d[:, 0], xt3)
    return jnp.transpose(
        out3.reshape(_B, x.shape[2], x.shape[3], ch), (0, 3, 1, 2)
    )


# speedup vs baseline: 2.1604x; 2.1604x over previous
import functools

import jax
import jax.numpy as jnp
from jax import lax
from jax.experimental import pallas as pl
from jax.experimental.pallas import tpu as pltpu
from jax.experimental.pallas import tpu_sc as plsc

_B = 32
_N = 192 * 56 * 56
_K = max(1, int(_N * 0.2))
_R = _N // 128
_HW = 56 * 56
_C = 192
_CPW = _C // 16
_CHR = 32
_NPAIR = _HW // (2 * _CHR)
_SIGN = 0x7FFFFFFF
_NC = 2
_NS = 16


def _sc_thresholds_kernel(x_hbm, thr_hbm, buf0, buf1, hist, thrv, gs, sem0, sem1):
    wid = lax.axis_index("s") * _NC + lax.axis_index("c")
    lane = lax.iota(jnp.int32, 16)
    ones = jnp.ones((16,), jnp.int32)

    def zero_hist(nwords):
        @plsc.parallel_loop(0, nwords, step=16, unroll=8)
        def _(j):
            hist[pl.ds(j, 16)] = jnp.zeros((16,), jnp.int32)

    def stream_pass(process):
        pltpu.async_copy(x_hbm.at[wid, pl.ds(0, _CHR)], buf0, sem0)
        pltpu.async_copy(x_hbm.at[wid, pl.ds(_CHR, _CHR)], buf1, sem1)

        def pbody(i, carry):
            nxt = (2 * i + 2) * _CHR
            pltpu.make_async_copy(x_hbm.at[wid, pl.ds(0, _CHR)], buf0, sem0).wait()
            process(buf0)

            @pl.when(i < _NPAIR - 1)
            def _():
                pltpu.async_copy(x_hbm.at[wid, pl.ds(nxt, _CHR)], buf0, sem0)

            pltpu.make_async_copy(x_hbm.at[wid, pl.ds(0, _CHR)], buf1, sem1).wait()
            process(buf1)

            @pl.when(i < _NPAIR - 1)
            def _():
                pltpu.async_copy(x_hbm.at[wid, pl.ds(nxt + _CHR, _CHR)], buf1, sem1)

            return carry

        lax.fori_loop(0, _NPAIR, pbody, 0)

    def hist_pass(make_idx_mask):
        def process(buf):
            @plsc.parallel_loop(0, _CHR, step=1, unroll=4)
            def _(r):
                for c in range(_CPW):
                    v = buf[r, pl.ds(c * 16, 16)]
                    t = plsc.bitcast(v, jnp.int32) & _SIGN
                    idx, m = make_idx_mask(t)
                    plsc.addupdate_scatter(hist, [idx], ones, mask=m)

        stream_pass(process)

    def select(ngroups, rank):
        def gbody(g, c):
            acc = hist[pl.ds(g * 256, 16)]
            for l in range(1, 16):
                acc = acc + hist[pl.ds(g * 256 + l * 16, 16)]
            gs[g] = jnp.sum(acc)
            return c

        lax.fori_loop(0, ngroups, gbody, 0)

        def sbody(i, carry):
            acc, grp, s_after = carry
            g = ngroups - 1 - i
            nacc = acc + gs[g]
            hit = jnp.logical_and(nacc >= rank, acc < rank)
            return (
                nacc,
                jnp.where(hit, g, grp),
                jnp.where(hit, acc, s_after),
            )

        _, grp, s_after = lax.fori_loop(
            0, ngroups, sbody, (jnp.int32(0), jnp.int32(0), jnp.int32(0))
        )

        tv = jnp.zeros((16,), jnp.int32)
        gbase = grp * 256
        for l in range(16):
            tv = tv + plsc.load_gather(hist, [gbase + lane * 16 + l])

        rr = rank - s_after
        rv = lax.rev(tv, (0,))
        cum = plsc.cumsum(rv)
        i0 = plsc.all_reduce_ffs(cum >= rr)
        sel = lane == i0
        rv_i = jnp.sum(jnp.where(sel, rv, 0))
        c_i = jnp.sum(jnp.where(sel, cum, 0))
        binlow = 15 - jnp.sum(jnp.where(sel, lane, 0))
        b = grp * 16 + binlow
        return b, rr - (c_i - rv_i)

    zero_hist(65536)

    def l1(t):
        return ((t >> 15) & 0x1FFF0) | lane, None

    hist_pass(l1)
    b1, r2 = select(256, jnp.int32(_K))

    zero_hist(65536)

    def l2(t):
        return ((t >> 3) & 0xFFF0) | lane, (t >> 19) == b1

    hist_pass(l2)
    b2, r3 = select(256, r2)

    p25 = (b1 << 12) | b2
    zero_hist(2048)

    def l3(t):
        return ((t & 0x7F) << 4) | lane, (t >> 7) == p25

    hist_pass(l3)
    b3, _ = select(8, r3)

    bits = (b1 << 19) | (b2 << 7) | b3
    thrv[...] = jnp.full((16,), 1, jnp.int32) * bits
    pltpu.sync_copy(thrv, thr_hbm.at[wid])


def _mask_kernel(thr_ref, x_ref, o_ref):
    b = thr_ref[pl.program_id(0)]
    xb = x_ref[0]
    bits = lax.bitcast_convert_type(xb, jnp.int32) & _SIGN
    o_ref[0] = jnp.where(bits >= b, xb, 0.0)


def kernel(x):
    hw = x.shape[2] * x.shape[3]
    ch = x.shape[1]
    xt3 = jnp.transpose(x, (0, 2, 3, 1)).reshape(_B, hw, ch)

    mesh = plsc.VectorSubcoreMesh(core_axis_name="c", subcore_axis_name="s")
    thr2d = pl.kernel(
        _sc_thresholds_kernel,
        out_type=jax.ShapeDtypeStruct((_B, 16), jnp.int32),
        mesh=mesh,
        compiler_params=pltpu.CompilerParams(needs_layout_passes=False),
        scratch_types=[
            pltpu.VMEM((_CHR, _C), jnp.float32),
            pltpu.VMEM((_CHR, _C), jnp.float32),
            pltpu.VMEM((65536,), jnp.int32),
            pltpu.VMEM((16,), jnp.int32),
            pltpu.SMEM((256,), jnp.int32),
            pltpu.SemaphoreType.DMA,
            pltpu.SemaphoreType.DMA,
        ],
    )(xt3)
    out3 = pl.pallas_call(
        _mask_kernel,
        grid=(_B,),
        in_specs=[
            pl.BlockSpec(memory_space=pltpu.SMEM),
            pl.BlockSpec((1, hw, ch), lambda i: (i, 0, 0)),
        ],
        out_specs=pl.BlockSpec((1, hw, ch), lambda i: (i, 0, 0)),
        out_shape=jax.ShapeDtypeStruct((_B, hw, ch), jnp.float32),
    )(thr2d[:, 0], xt3)
    return jnp.transpose(
        out3.reshape(_B, x.shape[2], x.shape[3], ch), (0, 3, 1, 2)
    )


# speedup vs baseline: 2.6208x; 1.2131x over previous
import functools

import jax
import jax.numpy as jnp
from jax import lax
from jax.experimental import pallas as pl
from jax.experimental.pallas import tpu as pltpu
from jax.experimental.pallas import tpu_sc as plsc

_B = 32
_N = 192 * 56 * 56
_K = max(1, int(_N * 0.2))
_R = _N // 128
_HW = 56 * 56
_C = 192
_CPW = _C // 16
_CHR = 32
_NPAIR = _HW // (2 * _CHR)
_SIGN = 0x7FFFFFFF
_NC = 2
_NS = 16


def _sc_thresholds_kernel(x_hbm, thr_hbm, buf0, buf1, hist, thrv, gs, sem0, sem1):
    wid = lax.axis_index("s") * _NC + lax.axis_index("c")
    lane = lax.iota(jnp.int32, 16)
    ones = jnp.ones((16,), jnp.int32)

    def zero_hist(nwords):
        @plsc.parallel_loop(0, nwords, step=16, unroll=8)
        def _(j):
            hist[pl.ds(j, 16)] = jnp.zeros((16,), jnp.int32)

    def stream_pass(process):
        pltpu.async_copy(x_hbm.at[wid, pl.ds(0, _CHR)], buf0, sem0)
        pltpu.async_copy(x_hbm.at[wid, pl.ds(_CHR, _CHR)], buf1, sem1)

        def pbody(i, carry):
            nxt = (2 * i + 2) * _CHR
            pltpu.make_async_copy(x_hbm.at[wid, pl.ds(0, _CHR)], buf0, sem0).wait()
            process(buf0)

            @pl.when(i < _NPAIR - 1)
            def _():
                pltpu.async_copy(x_hbm.at[wid, pl.ds(nxt, _CHR)], buf0, sem0)

            pltpu.make_async_copy(x_hbm.at[wid, pl.ds(0, _CHR)], buf1, sem1).wait()
            process(buf1)

            @pl.when(i < _NPAIR - 1)
            def _():
                pltpu.async_copy(x_hbm.at[wid, pl.ds(nxt + _CHR, _CHR)], buf1, sem1)

            return carry

        lax.fori_loop(0, _NPAIR, pbody, 0)

    def hist_pass(make_idx_mask):
        def process(buf):
            @plsc.parallel_loop(0, _CHR, step=1, unroll=4)
            def _(r):
                for c in range(_CPW):
                    v = buf[r, pl.ds(c * 16, 16)]
                    t = plsc.bitcast(v, jnp.int32) & _SIGN
                    idx, m = make_idx_mask(t)
                    cnt, lastm = plsc.scan_count(idx, mask=m)
                    plsc.addupdate_scatter(hist, [idx], cnt, mask=lastm)

        stream_pass(process)

    def select(nbins, rank):
        nsg = nbins // 256

        def gbody(g, c):
            acc = hist[pl.ds(g * 256, 16)]
            for l in range(1, 16):
                acc = acc + hist[pl.ds(g * 256 + l * 16, 16)]
            gs[g] = jnp.sum(acc)
            return c

        lax.fori_loop(0, nsg, gbody, 0)

        def sbody(i, carry):
            acc, grp, s_after = carry
            g = nsg - 1 - i
            nacc = acc + gs[g]
            hit = jnp.logical_and(nacc >= rank, acc < rank)
            return (
                nacc,
                jnp.where(hit, g, grp),
                jnp.where(hit, acc, s_after),
            )

        _, grp, s_after = lax.fori_loop(
            0, nsg, sbody, (jnp.int32(0), jnp.int32(0), jnp.int32(0))
        )
        rr = rank - s_after

        def vbody(i, carry):
            acc, vj, s2 = carry
            j = 15 - i
            v = hist[pl.ds(grp * 256 + j * 16, 16)]
            nacc = acc + jnp.sum(v)
            hit = jnp.logical_and(nacc >= rr, acc < rr)
            return (
                nacc,
                jnp.where(hit, j, vj),
                jnp.where(hit, acc, s2),
            )

        _, vj, s2 = lax.fori_loop(
            0, 16, vbody, (jnp.int32(0), jnp.int32(0), jnp.int32(0))
        )
        rr2 = rr - s2

        tv = hist[pl.ds(grp * 256 + vj * 16, 16)]
        rv = lax.rev(tv, (0,))
        cum = plsc.cumsum(rv)
        i0 = plsc.all_reduce_ffs(cum >= rr2)
        sel = lane == i0
        rv_i = jnp.sum(jnp.where(sel, rv, 0))
        c_i = jnp.sum(jnp.where(sel, cum, 0))
        binlow = 15 - jnp.sum(jnp.where(sel, lane, 0))
        b = grp * 256 + vj * 16 + binlow
        return b, rr2 - (c_i - rv_i)

    zero_hist(65536)
    hist_pass(lambda t: (t >> 15, None))
    b1, r2 = select(65536, jnp.int32(_K))

    zero_hist(32768)
    hist_pass(lambda t: (t & 0x7FFF, (t >> 15) == b1))
    b2, _ = select(32768, r2)

    bits = (b1 << 15) | b2
    thrv[...] = jnp.full((16,), 1, jnp.int32) * bits
    pltpu.sync_copy(thrv, thr_hbm.at[wid])


def _mask_kernel(thr_ref, x_ref, o_ref):
    b = thr_ref[pl.program_id(0)]
    xb = x_ref[0]
    bits = lax.bitcast_convert_type(xb, jnp.int32) & _SIGN
    o_ref[0] = jnp.where(bits >= b, xb, 0.0)


def kernel(x):
    hw = x.shape[2] * x.shape[3]
    ch = x.shape[1]
    xt3 = jnp.transpose(x, (0, 2, 3, 1)).reshape(_B, hw, ch)

    mesh = plsc.VectorSubcoreMesh(core_axis_name="c", subcore_axis_name="s")
    thr2d = pl.kernel(
        _sc_thresholds_kernel,
        out_type=jax.ShapeDtypeStruct((_B, 16), jnp.int32),
        mesh=mesh,
        compiler_params=pltpu.CompilerParams(needs_layout_passes=False),
        scratch_types=[
            pltpu.VMEM((_CHR, _C), jnp.float32),
            pltpu.VMEM((_CHR, _C), jnp.float32),
            pltpu.VMEM((65536,), jnp.int32),
            pltpu.VMEM((16,), jnp.int32),
            pltpu.SMEM((256,), jnp.int32),
            pltpu.SemaphoreType.DMA,
            pltpu.SemaphoreType.DMA,
        ],
    )(xt3)
    out3 = pl.pallas_call(
        _mask_kernel,
        grid=(_B,),
        in_specs=[
            pl.BlockSpec(memory_space=pltpu.SMEM),
            pl.BlockSpec((1, hw, ch), lambda i: (i, 0, 0)),
        ],
        out_specs=pl.BlockSpec((1, hw, ch), lambda i: (i, 0, 0)),
        out_shape=jax.ShapeDtypeStruct((_B, hw, ch), jnp.float32),
    )(thr2d[:, 0], xt3)
    return jnp.transpose(
        out3.reshape(_B, x.shape[2], x.shape[3], ch), (0, 3, 1, 2)
    )


# speedup vs baseline: 2.6255x; 1.0018x over previous
import jax
import jax.numpy as jnp
from jax import lax
from jax.experimental import pallas as pl
from jax.experimental.pallas import tpu as pltpu
from jax.experimental.pallas import tpu_sc as plsc

_B = 32
_N = 192 * 56 * 56
_K = max(1, int(_N * 0.2))
_HW = 56 * 56
_C = 192
_CPW = _C // 16
_CHR = 32
_NPAIR = _HW // (2 * _CHR)
_SIGN = 0x7FFFFFFF
_NC = 2
_NS = 16


def _sc_thresholds_kernel(x_hbm, thr_hbm, buf0, buf1, hist, thrv, gs, sem0, sem1):
    wid = lax.axis_index("s") * _NC + lax.axis_index("c")
    lane = lax.iota(jnp.int32, 16)

    def zero_hist(nwords):
        @plsc.parallel_loop(0, nwords, step=16, unroll=8)
        def _(j):
            hist[pl.ds(j, 16)] = jnp.zeros((16,), jnp.int32)

    def stream_pass(process):
        pltpu.async_copy(x_hbm.at[wid, pl.ds(0, _CHR)], buf0, sem0)
        pltpu.async_copy(x_hbm.at[wid, pl.ds(_CHR, _CHR)], buf1, sem1)

        def pbody(i, carry):
            nxt = (2 * i + 2) * _CHR
            pltpu.make_async_copy(x_hbm.at[wid, pl.ds(0, _CHR)], buf0, sem0).wait()
            process(buf0)

            @pl.when(i < _NPAIR - 1)
            def _():
                pltpu.async_copy(x_hbm.at[wid, pl.ds(nxt, _CHR)], buf0, sem0)

            pltpu.make_async_copy(x_hbm.at[wid, pl.ds(0, _CHR)], buf1, sem1).wait()
            process(buf1)

            @pl.when(i < _NPAIR - 1)
            def _():
                pltpu.async_copy(x_hbm.at[wid, pl.ds(nxt + _CHR, _CHR)], buf1, sem1)

            return carry

        lax.fori_loop(0, _NPAIR, pbody, 0)

    def hist_pass(make_idx_mask):
        def process(buf):
            @plsc.parallel_loop(0, _CHR, step=1, unroll=4)
            def _(r):
                for c in range(_CPW):
                    v = buf[r, pl.ds(c * 16, 16)]
                    t = plsc.bitcast(v, jnp.int32) & _SIGN
                    idx, m = make_idx_mask(t)
                    cnt, lastm = plsc.scan_count(idx, mask=m)
                    plsc.addupdate_scatter(hist, [idx], cnt, mask=lastm)

        stream_pass(process)

    def select(nbins, rank):
        nsg = nbins // 256

        def gbody(g, c):
            acc = hist[pl.ds(g * 256, 16)]
            for l in range(1, 16):
                acc = acc + hist[pl.ds(g * 256 + l * 16, 16)]
            gs[g] = jnp.sum(acc)
            return c

        lax.fori_loop(0, nsg, gbody, 0)

        def sbody(i, carry):
            acc, grp, s_after = carry
            g = nsg - 1 - i
            nacc = acc + gs[g]
            hit = jnp.logical_and(nacc >= rank, acc < rank)
            return (
                nacc,
                jnp.where(hit, g, grp),
                jnp.where(hit, acc, s_after),
            )

        _, grp, s_after = lax.fori_loop(
            0, nsg, sbody, (jnp.int32(0), jnp.int32(0), jnp.int32(0))
        )
        rr = rank - s_after

        def vbody(i, carry):
            acc, vj, s2 = carry
            j = 15 - i
            v = hist[pl.ds(grp * 256 + j * 16, 16)]
            nacc = acc + jnp.sum(v)
            hit = jnp.logical_and(nacc >= rr, acc < rr)
            return (
                nacc,
                jnp.where(hit, j, vj),
                jnp.where(hit, acc, s2),
            )

        _, vj, s2 = lax.fori_loop(
            0, 16, vbody, (jnp.int32(0), jnp.int32(0), jnp.int32(0))
        )
        rr2 = rr - s2

        tv = hist[pl.ds(grp * 256 + vj * 16, 16)]
        rv = lax.rev(tv, (0,))
        cum = plsc.cumsum(rv)
        i0 = plsc.all_reduce_ffs(cum >= rr2)
        sel = lane == i0
        rv_i = jnp.sum(jnp.where(sel, rv, 0))
        c_i = jnp.sum(jnp.where(sel, cum, 0))
        binlow = 15 - jnp.sum(jnp.where(sel, lane, 0))
        b = grp * 256 + vj * 16 + binlow
        return b, rr2 - (c_i - rv_i)

    zero_hist(65536)
    hist_pass(lambda t: (t >> 15, None))
    b1, r2 = select(65536, jnp.int32(_K))

    zero_hist(32768)
    hist_pass(lambda t: (t & 0x7FFF, (t >> 15) == b1))
    b2, _ = select(32768, r2)

    bits = (b1 << 15) | b2
    thrv[...] = jnp.full((16,), 1, jnp.int32) * bits
    pltpu.sync_copy(thrv, thr_hbm.at[wid])


def _mask_kernel(thr_ref, x_ref, o_ref):
    b = thr_ref[pl.program_id(0)]
    xb = x_ref[0]
    bits = lax.bitcast_convert_type(xb, jnp.int32) & _SIGN
    o_ref[0] = jnp.where(bits >= b, xb, 0.0)


def kernel(x):
    hw = x.shape[2] * x.shape[3]
    ch = x.shape[1]
    xt3 = jnp.transpose(x, (0, 2, 3, 1)).reshape(_B, hw, ch)

    mesh = plsc.VectorSubcoreMesh(core_axis_name="c", subcore_axis_name="s")
    thr2d = pl.kernel(
        _sc_thresholds_kernel,
        out_type=jax.ShapeDtypeStruct((_B, 16), jnp.int32),
        mesh=mesh,
        compiler_params=pltpu.CompilerParams(needs_layout_passes=False),
        scratch_types=[
            pltpu.VMEM((_CHR, _C), jnp.float32),
            pltpu.VMEM((_CHR, _C), jnp.float32),
            pltpu.VMEM((65536,), jnp.int32),
            pltpu.VMEM((16,), jnp.int32),
            pltpu.SMEM((256,), jnp.int32),
            pltpu.SemaphoreType.DMA,
            pltpu.SemaphoreType.DMA,
        ],
    )(xt3)
    out3 = pl.pallas_call(
        _mask_kernel,
        grid=(_B,),
        in_specs=[
            pl.BlockSpec(memory_space=pltpu.SMEM),
            pl.BlockSpec((1, hw, ch), lambda i: (i, 0, 0)),
        ],
        out_specs=pl.BlockSpec((1, hw, ch), lambda i: (i, 0, 0)),
        out_shape=jax.ShapeDtypeStruct((_B, hw, ch), jnp.float32),
    )(thr2d[:, 0], xt3)
    return jnp.transpose(
        out3.reshape(_B, x.shape[2], x.shape[3], ch), (0, 3, 1, 2)
    )
